# baseline (speedup 1.0000x reference)
import functools

import jax
import jax.numpy as jnp
from jax import lax
from jax.experimental import pallas as pl
from jax.experimental.pallas import tpu as pltpu
from jax.experimental.pallas import tpu_sc as plsc

EMBED = 32
OUT_D = 3 * EMBED
PAD_D = 128
N0, N1, N2 = 22, 6, 2
NT = N0 * N1 * N2
E_TOTAL = 1600000

NC, NS, L = 2, 16, 16
NW = NC * NS
CHUNK = 256
NGRP = CHUNK // L
NCH_TOT = E_TOTAL // CHUNK
NCH_BASE = NCH_TOT // NW
NCH_REM = NCH_TOT - NCH_BASE * NW


def _build_table(W0, W1, W2):
    def body(w0_ref, w1_ref, w2_ref, t_ref):
        i = lax.broadcasted_iota(jnp.int32, (NT, 1), 0)
        oh0 = (i // (N1 * N2) == lax.broadcasted_iota(jnp.int32, (NT, N0), 1))
        oh1 = ((i // N2) % N1 == lax.broadcasted_iota(jnp.int32, (NT, N1), 1))
        oh2 = (i % N2 == lax.broadcasted_iota(jnp.int32, (NT, N2), 1))
        t0 = jnp.dot(oh0.astype(jnp.float32), w0_ref[:],
                     preferred_element_type=jnp.float32,
                     precision=lax.Precision.HIGHEST)
        t1 = jnp.dot(oh1.astype(jnp.float32), w1_ref[:],
                     preferred_element_type=jnp.float32,
                     precision=lax.Precision.HIGHEST)
        t2 = jnp.dot(oh2.astype(jnp.float32), w2_ref[:],
                     preferred_element_type=jnp.float32,
                     precision=lax.Precision.HIGHEST)
        pad = jnp.zeros((NT, PAD_D - OUT_D), jnp.float32)
        t_ref[:] = jnp.concatenate([t0, t1, t2, pad], axis=1)

    return pl.pallas_call(
        body,
        out_shape=jax.ShapeDtypeStruct((NT, PAD_D), jnp.float32),
    )(W0, W1, W2)


_mesh = plsc.VectorSubcoreMesh(core_axis_name="c", subcore_axis_name="s")


@functools.partial(
    pl.kernel,
    out_type=jax.ShapeDtypeStruct((OUT_D, E_TOTAL), jnp.float32),
    mesh=_mesh,
    compiler_params=pltpu.CompilerParams(needs_layout_passes=False),
    scratch_types=[
        pltpu.VMEM((2, 1, CHUNK), jnp.int32),
        pltpu.VMEM((2, 1, CHUNK), jnp.int32),
        pltpu.VMEM((2, 1, CHUNK), jnp.int32),
        pltpu.VMEM((NT * PAD_D,), jnp.float32),
        pltpu.VMEM((2, OUT_D, CHUNK), jnp.float32),
        pltpu.SemaphoreType.DMA,
        pltpu.SemaphoreType.DMA,
        pltpu.SemaphoreType.DMA,
        pltpu.SemaphoreType.DMA,
    ],
)
def _sc_gather(a0_hbm, a1_hbm, a2_hbm, tflat_hbm, out_hbm,
               a0_v, a1_v, a2_v, t_v, rows_v,
               asem0, asem1, wsem0, wsem1):
    wid = lax.axis_index("s") * NC + lax.axis_index("c")
    base_chunk = NCH_BASE * wid + jnp.minimum(wid, NCH_REM)

    pltpu.sync_copy(tflat_hbm, t_v)

    asem = (asem0, asem1)
    wsem = (wsem0, wsem1)

    def ebase(k):
        return pl.multiple_of((base_chunk + k) * CHUNK, 128)

    def attr_copies(k, p):
        base = ebase(k)
        srcs = (a0_hbm, a1_hbm, a2_hbm)
        dsts = (a0_v, a1_v, a2_v)
        return [pltpu.make_async_copy(
            srcs[f].at[pl.ds(base, CHUNK)], dsts[f].at[p, 0], asem[p])
            for f in range(3)]

    def write_copy(k, p):
        base = ebase(k)
        return pltpu.make_async_copy(
            rows_v.at[p], out_hbm.at[:, pl.ds(base, CHUNK)], wsem[p])

    def transpose_chunk(p):
        def group_body(g, carry):
            col = g * L
            v0 = jnp.minimum(a0_v[p, 0, pl.ds(col, L)], N0 - 1)
            v1 = jnp.minimum(a1_v[p, 0, pl.ds(col, L)], N1 - 1)
            v2 = jnp.minimum(a2_v[p, 0, pl.ds(col, L)], N2 - 1)
            flat = (v0 * (N1 * N2) + v1 * N2 + v2) * PAD_D
            for d in range(OUT_D):
                rows_v[p, d, pl.ds(col, L)] = plsc.load_gather(
                    t_v, [flat + d])
            return carry

        lax.fori_loop(0, NGRP, group_body, jnp.int32(0))

    def process(k, p, prefetch_next, first_pair, prefetched=True):
        if not prefetched:
            for cp in attr_copies(k, p):
                cp.start()
        for cp in attr_copies(k, p):
            cp.wait()
        if prefetch_next:
            for cp in attr_copies(k + 1, 1 - p):
                cp.start()
        if not first_pair:
            write_copy(k, p).wait()
        transpose_chunk(p)
        write_copy(k, p).start()

    for _cp in attr_copies(0, 0):
        _cp.start()

    process(0, 0, True, True)
    process(1, 1, True, True)

    def pair_body(k2, carry):
        k = 2 * k2
        process(k, 0, True, False)
        process(k + 1, 1, True, False)
        return carry

    lax.fori_loop(1, NCH_BASE // 2, pair_body, jnp.int32(0))

    process(NCH_BASE - 1, 0, False, False)

    @pl.when(wid < NCH_REM)
    def _extra_chunk():
        process(NCH_BASE, 1, False, False, prefetched=False)
        write_copy(NCH_BASE, 1).wait()

    @pl.when(wid >= NCH_REM)
    def _drain_last_odd():
        write_copy(NCH_BASE - 2, 1).wait()

    write_copy(NCH_BASE - 1, 0).wait()


def kernel(edge_attr, W0, W1, W2):
    table = _build_table(W0, W1, W2).reshape(-1)
    a0 = edge_attr[:, 0]
    a1 = edge_attr[:, 1]
    a2 = edge_attr[:, 2]
    out_t = _sc_gather(a0, a1, a2, table)
    return out_t.T


# speedup vs baseline: 2.5878x; 2.5878x over previous
import functools

import jax
import jax.numpy as jnp
from jax import lax
from jax.experimental import pallas as pl
from jax.experimental.pallas import tpu as pltpu
from jax.experimental.pallas import tpu_sc as plsc

EMBED = 32
OUT_D = 3 * EMBED
PAD_D = 128
N0, N1, N2 = 22, 6, 2
NT = N0 * N1 * N2
E_TOTAL = 1600000

NC, NS, L = 2, 16, 16
NW = NC * NS
PER_W = E_TOTAL // NW
CHUNK = 400
NGRP = CHUNK // L
NSEG = 5
SEG = CHUNK // NSEG
NCHUNK = PER_W // CHUNK
NK = OUT_D // L


def _build_table(W0, W1, W2):
    def body(w0_ref, w1_ref, w2_ref, t_ref):
        i = lax.broadcasted_iota(jnp.int32, (NT, 1), 0)
        oh0 = (i // (N1 * N2) == lax.broadcasted_iota(jnp.int32, (NT, N0), 1))
        oh1 = ((i // N2) % N1 == lax.broadcasted_iota(jnp.int32, (NT, N1), 1))
        oh2 = (i % N2 == lax.broadcasted_iota(jnp.int32, (NT, N2), 1))
        t0 = jnp.dot(oh0.astype(jnp.float32), w0_ref[:],
                     preferred_element_type=jnp.float32,
                     precision=lax.Precision.HIGHEST)
        t1 = jnp.dot(oh1.astype(jnp.float32), w1_ref[:],
                     preferred_element_type=jnp.float32,
                     precision=lax.Precision.HIGHEST)
        t2 = jnp.dot(oh2.astype(jnp.float32), w2_ref[:],
                     preferred_element_type=jnp.float32,
                     precision=lax.Precision.HIGHEST)
        pad = jnp.zeros((NT, PAD_D - OUT_D), jnp.float32)
        t_ref[:] = jnp.concatenate([t0, t1, t2, pad], axis=1)

    return pl.pallas_call(
        body,
        out_shape=jax.ShapeDtypeStruct((NT, PAD_D), jnp.float32),
    )(W0, W1, W2)


_mesh = plsc.VectorSubcoreMesh(core_axis_name="c", subcore_axis_name="s")


@functools.partial(
    pl.kernel,
    out_type=jax.ShapeDtypeStruct((E_TOTAL, OUT_D), jnp.float32),
    mesh=_mesh,
    scratch_types=[
        pltpu.VMEM((2, 1, CHUNK), jnp.int32),
        pltpu.VMEM((2, 1, CHUNK), jnp.int32),
        pltpu.VMEM((2, 1, CHUNK), jnp.int32),
        pltpu.VMEM((2, NSEG, 1, SEG), jnp.int32),
        pltpu.VMEM((2, SEG, PAD_D), jnp.float32),
        pltpu.VMEM((2, CHUNK, OUT_D), jnp.float32),
        pltpu.VMEM_SHARED((NT, PAD_D), jnp.float32),
        pltpu.SemaphoreType.DMA,
        pltpu.SemaphoreType.DMA,
        pltpu.SemaphoreType.DMA,
        pltpu.SemaphoreType.DMA,
        pltpu.SemaphoreType.DMA,
        pltpu.SemaphoreType.DMA,
    ],
)
def _sc_gather(a0_hbm, a1_hbm, a2_hbm, t_hbm, out_hbm,
               a0_v, a1_v, a2_v, idx_v, seg_v, rows_v, t_sh,
               asem0, asem1, gsem0, gsem1, wsem0, wsem1):
    wid = lax.axis_index("s") * NC + lax.axis_index("c")
    base0 = wid * PER_W

    @pl.when(lax.axis_index("s") == 0)
    def _stage_table():
        pltpu.sync_copy(t_hbm, t_sh)

    plsc.subcore_barrier()

    asem = (asem0, asem1)
    gsem = (gsem0, gsem1)
    wsem = (wsem0, wsem1)

    def attr_copies(k, p):
        base = pl.multiple_of(base0 + k * CHUNK, 16)
        srcs = (a0_hbm, a1_hbm, a2_hbm)
        dsts = (a0_v, a1_v, a2_v)
        return [pltpu.make_async_copy(
            srcs[f].at[pl.ds(base, CHUNK)], dsts[f].at[p, 0], asem[p])
            for f in range(3)]

    def gather_copy(p, s):
        u = s % 2
        return pltpu.make_async_copy(
            t_sh.at[idx_v.at[p, s, 0]], seg_v.at[u], gsem[u])

    def write_copy(k, p):
        base = pl.multiple_of(base0 + k * CHUNK, 16)
        return pltpu.make_async_copy(
            rows_v.at[p], out_hbm.at[pl.ds(base, CHUNK)], wsem[p])

    def compute_idx(p):
        for g in range(NGRP):
            s, col = divmod(g * L, SEG)
            v0 = jnp.minimum(a0_v[p, 0, pl.ds(g * L, L)], N0 - 1)
            v1 = jnp.minimum(a1_v[p, 0, pl.ds(g * L, L)], N1 - 1)
            v2 = jnp.minimum(a2_v[p, 0, pl.ds(g * L, L)], N2 - 1)
            idx_v[p, s, 0, pl.ds(col, L)] = v0 * (N1 * N2) + v1 * N2 + v2

    def repack_seg(p, s):
        u = s % 2

        def row_body(r2, carry):
            r = 2 * r2
            vals = [seg_v[u, r + j, pl.ds(k * L, L)]
                    for j in range(2) for k in range(NK)]
            for j in range(2):
                for k in range(NK):
                    rows_v[p, s * SEG + r + j, pl.ds(k * L, L)] = (
                        vals[j * NK + k])
            return carry

        lax.fori_loop(0, SEG // 2, row_body, jnp.int32(0))

    def process(k, p, prefetch_next, first_pair):
        for cp in attr_copies(k, p):
            cp.wait()
        compute_idx(p)
        if prefetch_next:
            for cp in attr_copies(k + 1, 1 - p):
                cp.start()
        if not first_pair:
            write_copy(k, p).wait()
        gather_copy(p, 0).start()
        gather_copy(p, 1).start()
        for s in range(NSEG):
            gather_copy(p, s).wait()
            repack_seg(p, s)
            if s + 2 < NSEG:
                gather_copy(p, s + 2).start()
        write_copy(k, p).start()

    for _cp in attr_copies(0, 0):
        _cp.start()

    process(0, 0, True, True)
    process(1, 1, True, True)

    def pair_body(k2, carry):
        k = 2 * k2
        process(k, 0, True, False)
        process(k + 1, 1, True, False)
        return carry

    lax.fori_loop(1, NCHUNK // 2, pair_body, jnp.int32(0))

    k_last = NCHUNK - 1
    process(k_last, 0, False, False)
    write_copy(k_last - 1, 1).wait()
    write_copy(k_last, 0).wait()


def kernel(edge_attr, W0, W1, W2):
    table = _build_table(W0, W1, W2)
    a0 = edge_attr[:, 0]
    a1 = edge_attr[:, 1]
    a2 = edge_attr[:, 2]
    return _sc_gather(a0, a1, a2, table)
